# Initial kernel scaffold; baseline (speedup 1.0000x reference)
import jax
import jax.numpy as jnp
from jax import lax
from jax.experimental import pallas as pl
from jax.experimental.pallas import tpu as pltpu
from jax.experimental.pallas import tpu_sc as plsc

f32 = jnp.float32
i32 = jnp.int32

D = 128
NP = 10240
NC, NS, L = 2, 16, 16
NW = NC * NS
EC = 81
EPW = EC * 128
EP = NW * EPW
BR = 1024
GRID = NP // BR
RPT = NP // NS


def _mm_att_body(x_ref, w_ref, av_ref, h_ref, as_ref, ad_ref):
    h = jnp.dot(x_ref[...], w_ref[...], preferred_element_type=f32)
    h_ref[...] = h
    as_ref[...] = jnp.sum(h * av_ref[0:1], axis=1).reshape(1, 1, BR)
    ad_ref[...] = jnp.sum(h * av_ref[1:2], axis=1).reshape(1, 1, BR)


_mm_att = pl.pallas_call(
    _mm_att_body,
    grid=(GRID,),
    in_specs=[
        pl.BlockSpec((BR, D), lambda i: (i, 0)),
        pl.BlockSpec((D, D), lambda i: (0, 0)),
        pl.BlockSpec((2, D), lambda i: (0, 0)),
    ],
    out_specs=[
        pl.BlockSpec((BR, D), lambda i: (i, 0)),
        pl.BlockSpec((1, 1, BR), lambda i: (i, 0, 0)),
        pl.BlockSpec((1, 1, BR), lambda i: (i, 0, 0)),
    ],
    out_shape=[
        jax.ShapeDtypeStruct((NP, D), f32),
        jax.ShapeDtypeStruct((GRID, 1, BR), f32),
        jax.ShapeDtypeStruct((GRID, 1, BR), f32),
    ],
)


def _comb_mm_body(op_ref, dp_ref, b_ref, w_ref, av_ref, h_ref, as_ref, ad_ref):
    z = op_ref[0] + op_ref[1]
    dtot = jnp.sum(dp_ref[...], axis=0)
    v = z / (dtot[:, None] + 1e-16) + b_ref[...]
    v = jnp.where(v > 0.0, v, jnp.exp(v) - 1.0)
    h = jnp.dot(v, w_ref[...], preferred_element_type=f32)
    h_ref[...] = h
    as_ref[...] = jnp.sum(h * av_ref[0:1], axis=1).reshape(1, 1, BR)
    ad_ref[...] = jnp.sum(h * av_ref[1:2], axis=1).reshape(1, 1, BR)


_comb_mm = pl.pallas_call(
    _comb_mm_body,
    grid=(GRID,),
    in_specs=[
        pl.BlockSpec((NC, BR, D), lambda i: (0, i, 0)),
        pl.BlockSpec((NW, BR), lambda i: (0, i)),
        pl.BlockSpec((1, D), lambda i: (0, 0)),
        pl.BlockSpec((D, D), lambda i: (0, 0)),
        pl.BlockSpec((2, D), lambda i: (0, 0)),
    ],
    out_specs=[
        pl.BlockSpec((BR, D), lambda i: (i, 0)),
        pl.BlockSpec((1, 1, BR), lambda i: (i, 0, 0)),
        pl.BlockSpec((1, 1, BR), lambda i: (i, 0, 0)),
    ],
    out_shape=[
        jax.ShapeDtypeStruct((NP, D), f32),
        jax.ShapeDtypeStruct((GRID, 1, BR), f32),
        jax.ShapeDtypeStruct((GRID, 1, BR), f32),
    ],
)


def _final_body(op_ref, dp_ref, b_ref, o_ref):
    z = op_ref[0] + op_ref[1]
    dtot = jnp.sum(dp_ref[...], axis=0)
    o_ref[...] = z / (dtot[:, None] + 1e-16) + b_ref[...]


_final = pl.pallas_call(
    _final_body,
    grid=(GRID,),
    in_specs=[
        pl.BlockSpec((NC, BR, D), lambda i: (0, i, 0)),
        pl.BlockSpec((NW, BR), lambda i: (0, i)),
        pl.BlockSpec((1, D), lambda i: (0, 0)),
    ],
    out_specs=pl.BlockSpec((BR, D), lambda i: (i, 0)),
    out_shape=jax.ShapeDtypeStruct((NP, D), f32),
)


def _sc_gat_body(h_hbm, as_hbm, ad_hbm, src_hbm, dst_hbm,
                 outp_hbm, denp_hbm,
                 as_v, ad_v, den_v, src_v, dst_v, ex_v, rows_v, acc_sh, sem):
    c = lax.axis_index("c")
    s = lax.axis_index("s")
    wid = c * NS + s

    pltpu.sync_copy(as_hbm, as_v)
    pltpu.sync_copy(ad_hbm, ad_v)
    pltpu.sync_copy(src_hbm.at[wid], src_v)
    pltpu.sync_copy(dst_hbm.at[wid], dst_v)

    zeros16 = jnp.zeros((L,), f32)

    def _zero_den(i, carry):
        den_v[pl.ds(i * L, L)] = zeros16
        return carry

    lax.fori_loop(0, NP // L, _zero_den, 0)

    def _zero_rows(i, carry):
        for k in range(D // L):
            rows_v[i, pl.ds(k * L, L)] = zeros16
        return carry

    lax.fori_loop(0, 128, _zero_rows, 0)

    for k in range(RPT // 128):
        pltpu.sync_copy(rows_v, acc_sh.at[pl.ds(s * RPT + k * 128, 128)])
    plsc.subcore_barrier()

    def _chunk(ci, carry):
        gcp = pltpu.async_copy(h_hbm.at[src_v.at[ci]], rows_v, sem)

        def _ex16(j, carry2):
            sl = pl.ds(j * L, L)
            sv = src_v[ci, sl]
            dv = dst_v[ci, sl]
            a = plsc.load_gather(as_v, [sv]) + plsc.load_gather(ad_v, [dv])
            a = jnp.where(a >= 0.0, a, 0.2 * a)
            e = jnp.exp(a)
            ex_v[pl.ds(ci * 128 + j * L, L)] = e
            plsc.addupdate_scatter(den_v, [dv], e)
            return carry2

        lax.fori_loop(0, 128 // L, _ex16, 0)
        gcp.wait()

        def _scale(j, carry2):
            es = plsc.load_gather(ex_v, [jnp.full((L,), ci * 128 + j, i32)])
            for k in range(D // L):
                sl = pl.ds(k * L, L)
                rows_v[j, sl] = rows_v[j, sl] * es
            return carry2

        lax.fori_loop(0, 128, _scale, 0)
        pltpu.sync_copy(rows_v, acc_sh.at[dst_v.at[ci]], add=True)
        return carry

    lax.fori_loop(0, EC, _chunk, 0)

    plsc.subcore_barrier()
    pltpu.sync_copy(den_v, denp_hbm.at[wid])
    base = s * RPT
    pltpu.sync_copy(acc_sh.at[pl.ds(base, RPT)],
                    outp_hbm.at[c, pl.ds(base, RPT)])


_sc_gat = pl.kernel(
    _sc_gat_body,
    out_type=[
        jax.ShapeDtypeStruct((NC, NP, D), f32),
        jax.ShapeDtypeStruct((NW, NP), f32),
    ],
    mesh=plsc.VectorSubcoreMesh(core_axis_name="c", subcore_axis_name="s",
                                num_cores=NC, num_subcores=NS),
    scratch_types=[
        pltpu.VMEM((NP,), f32),
        pltpu.VMEM((NP,), f32),
        pltpu.VMEM((NP,), f32),
        pltpu.VMEM((EC, 128), i32),
        pltpu.VMEM((EC, 128), i32),
        pltpu.VMEM((EPW,), f32),
        pltpu.VMEM((128, D), f32),
        pltpu.VMEM_SHARED((NP, D), f32),
        pltpu.SemaphoreType.DMA,
    ],
)


def kernel(x, edge_index, batch, W1, as1, ad1, b1, W2, as2, ad2, b2,
           W3, as3, ad3, b3):
    n, d = x.shape
    e = edge_index.shape[1]
    xp = jnp.pad(x, ((0, NP - n), (0, 0)))
    loop = jnp.arange(n, dtype=i32)
    pad_e = jnp.full((EP - e - n,), n + 1, i32)
    src = jnp.concatenate([edge_index[0], loop, pad_e]).reshape(NW, EC, 128)
    dst = jnp.concatenate([edge_index[1], loop, pad_e]).reshape(NW, EC, 128)

    h, asd, add_ = _mm_att(xp, W1, jnp.concatenate([as1, ad1], axis=0))
    outp, denp = _sc_gat(h, asd.reshape(NP), add_.reshape(NP), src, dst)

    h, asd, add_ = _comb_mm(outp, denp, b1.reshape(1, D), W2,
                            jnp.concatenate([as2, ad2], axis=0))
    outp, denp = _sc_gat(h, asd.reshape(NP), add_.reshape(NP), src, dst)

    h, asd, add_ = _comb_mm(outp, denp, b2.reshape(1, D), W3,
                            jnp.concatenate([as3, ad3], axis=0))
    outp, denp = _sc_gat(h, asd.reshape(NP), add_.reshape(NP), src, dst)

    out = _final(outp, denp, b3.reshape(1, D))
    return out[:n]


# speedup vs baseline: 18.0545x; 18.0545x over previous
import jax
import jax.numpy as jnp
from jax import lax
from jax.experimental import pallas as pl
from jax.experimental.pallas import tpu as pltpu
from jax.experimental.pallas import tpu_sc as plsc

f32 = jnp.float32
i32 = jnp.int32

D = 128
NP = 10240
NC, NS, L = 2, 16, 16
NW = NC * NS
CH = 64
EC = 162
EPW = EC * CH
EP = NW * EPW
BR = 1024
GRID = NP // BR
RPT = NP // NS


def _mm_att_body(x_ref, w_ref, av_ref, h_ref, as_ref, ad_ref):
    h = jnp.dot(x_ref[...], w_ref[...], preferred_element_type=f32)
    h_ref[...] = h
    as_ref[...] = jnp.sum(h * av_ref[0:1], axis=1).reshape(1, 1, BR)
    ad_ref[...] = jnp.sum(h * av_ref[1:2], axis=1).reshape(1, 1, BR)


_mm_att = pl.pallas_call(
    _mm_att_body,
    grid=(GRID,),
    in_specs=[
        pl.BlockSpec((BR, D), lambda i: (i, 0)),
        pl.BlockSpec((D, D), lambda i: (0, 0)),
        pl.BlockSpec((2, D), lambda i: (0, 0)),
    ],
    out_specs=[
        pl.BlockSpec((BR, D), lambda i: (i, 0)),
        pl.BlockSpec((1, 1, BR), lambda i: (i, 0, 0)),
        pl.BlockSpec((1, 1, BR), lambda i: (i, 0, 0)),
    ],
    out_shape=[
        jax.ShapeDtypeStruct((NP, D), f32),
        jax.ShapeDtypeStruct((GRID, 1, BR), f32),
        jax.ShapeDtypeStruct((GRID, 1, BR), f32),
    ],
)


def _comb_mm_body(op_ref, dp_ref, b_ref, w_ref, av_ref, h_ref, as_ref, ad_ref):
    z = op_ref[0] + op_ref[1]
    dtot = jnp.sum(dp_ref[...], axis=0)
    v = z / (dtot[:, None] + 1e-16) + b_ref[...]
    v = jnp.where(v > 0.0, v, jnp.exp(v) - 1.0)
    h = jnp.dot(v, w_ref[...], preferred_element_type=f32)
    h_ref[...] = h
    as_ref[...] = jnp.sum(h * av_ref[0:1], axis=1).reshape(1, 1, BR)
    ad_ref[...] = jnp.sum(h * av_ref[1:2], axis=1).reshape(1, 1, BR)


_comb_mm = pl.pallas_call(
    _comb_mm_body,
    grid=(GRID,),
    in_specs=[
        pl.BlockSpec((NC, BR, D), lambda i: (0, i, 0)),
        pl.BlockSpec((NW, BR), lambda i: (0, i)),
        pl.BlockSpec((1, D), lambda i: (0, 0)),
        pl.BlockSpec((D, D), lambda i: (0, 0)),
        pl.BlockSpec((2, D), lambda i: (0, 0)),
    ],
    out_specs=[
        pl.BlockSpec((BR, D), lambda i: (i, 0)),
        pl.BlockSpec((1, 1, BR), lambda i: (i, 0, 0)),
        pl.BlockSpec((1, 1, BR), lambda i: (i, 0, 0)),
    ],
    out_shape=[
        jax.ShapeDtypeStruct((NP, D), f32),
        jax.ShapeDtypeStruct((GRID, 1, BR), f32),
        jax.ShapeDtypeStruct((GRID, 1, BR), f32),
    ],
)


def _final_body(op_ref, dp_ref, b_ref, o_ref):
    z = op_ref[0] + op_ref[1]
    dtot = jnp.sum(dp_ref[...], axis=0)
    o_ref[...] = z / (dtot[:, None] + 1e-16) + b_ref[...]


_final = pl.pallas_call(
    _final_body,
    grid=(GRID,),
    in_specs=[
        pl.BlockSpec((NC, BR, D), lambda i: (0, i, 0)),
        pl.BlockSpec((NW, BR), lambda i: (0, i)),
        pl.BlockSpec((1, D), lambda i: (0, 0)),
    ],
    out_specs=pl.BlockSpec((BR, D), lambda i: (i, 0)),
    out_shape=jax.ShapeDtypeStruct((NP, D), f32),
)


def _sc_gat_body(h_hbm, as_hbm, ad_hbm, src_hbm, dst_hbm,
                 outp_hbm, denp_hbm,
                 as_v, ad_v, den_v, src_c, dst_c, ex_c, rows_v, acc_sh, sem):
    c = lax.axis_index("c")
    s = lax.axis_index("s")
    wid = c * NS + s

    pltpu.sync_copy(as_hbm, as_v)
    pltpu.sync_copy(ad_hbm, ad_v)

    zeros16 = jnp.zeros((L,), f32)

    def _zero_den(i, carry):
        den_v[pl.ds(i * L, L)] = zeros16
        return carry

    lax.fori_loop(0, NP // L, _zero_den, 0)

    def _zero_rows(i, carry):
        for k in range(D // L):
            rows_v[i, pl.ds(k * L, L)] = zeros16
        return carry

    lax.fori_loop(0, CH, _zero_rows, 0)

    for k in range(RPT // CH):
        pltpu.sync_copy(rows_v, acc_sh.at[pl.ds(s * RPT + k * CH, CH)])
    plsc.subcore_barrier()

    def _chunk(ci, carry):
        pltpu.sync_copy(src_hbm.at[wid, ci], src_c)
        pltpu.sync_copy(dst_hbm.at[wid, ci], dst_c)
        gcp = pltpu.async_copy(h_hbm.at[src_c], rows_v, sem)

        def _ex16(j, carry2):
            sl = pl.ds(j * L, L)
            sv = src_c[sl]
            dv = dst_c[sl]
            a = plsc.load_gather(as_v, [sv]) + plsc.load_gather(ad_v, [dv])
            a = jnp.where(a >= 0.0, a, 0.2 * a)
            e = jnp.exp(a)
            ex_c[sl] = e
            plsc.addupdate_scatter(den_v, [dv], e)
            return carry2

        lax.fori_loop(0, CH // L, _ex16, 0)
        gcp.wait()

        def _scale(j, carry2):
            es = plsc.load_gather(ex_c, [jnp.full((L,), j, i32)])
            for k in range(D // L):
                sl = pl.ds(k * L, L)
                rows_v[j, sl] = rows_v[j, sl] * es
            return carry2

        lax.fori_loop(0, CH, _scale, 0)
        pltpu.sync_copy(rows_v, acc_sh.at[dst_c], add=True)
        return carry

    lax.fori_loop(0, EC, _chunk, 0)

    plsc.subcore_barrier()
    pltpu.sync_copy(den_v, denp_hbm.at[wid])
    base = s * RPT
    pltpu.sync_copy(acc_sh.at[pl.ds(base, RPT)],
                    outp_hbm.at[c, pl.ds(base, RPT)])


_sc_gat = pl.kernel(
    _sc_gat_body,
    out_type=[
        jax.ShapeDtypeStruct((NC, NP, D), f32),
        jax.ShapeDtypeStruct((NW, NP), f32),
    ],
    mesh=plsc.VectorSubcoreMesh(core_axis_name="c", subcore_axis_name="s",
                                num_cores=NC, num_subcores=NS),
    compiler_params=pltpu.CompilerParams(needs_layout_passes=False),
    scratch_types=[
        pltpu.VMEM((NP,), f32),
        pltpu.VMEM((NP,), f32),
        pltpu.VMEM((NP,), f32),
        pltpu.VMEM((CH,), i32),
        pltpu.VMEM((CH,), i32),
        pltpu.VMEM((CH,), f32),
        pltpu.VMEM((CH, D), f32),
        pltpu.VMEM_SHARED((NP, D), f32),
        pltpu.SemaphoreType.DMA,
    ],
)


def kernel(x, edge_index, batch, W1, as1, ad1, b1, W2, as2, ad2, b2,
           W3, as3, ad3, b3):
    n, d = x.shape
    e = edge_index.shape[1]
    xp = jnp.pad(x, ((0, NP - n), (0, 0)))
    loop = jnp.arange(n, dtype=i32)
    pad_e = jnp.full((EP - e - n,), n + 1, i32)
    src = jnp.concatenate([edge_index[0], loop, pad_e]).reshape(NW, EC, CH)
    dst = jnp.concatenate([edge_index[1], loop, pad_e]).reshape(NW, EC, CH)

    h, asd, add_ = _mm_att(xp, W1, jnp.concatenate([as1, ad1], axis=0))
    outp, denp = _sc_gat(h, asd.reshape(NP), add_.reshape(NP), src, dst)

    h, asd, add_ = _comb_mm(outp, denp, b1.reshape(1, D), W2,
                            jnp.concatenate([as2, ad2], axis=0))
    outp, denp = _sc_gat(h, asd.reshape(NP), add_.reshape(NP), src, dst)

    h, asd, add_ = _comb_mm(outp, denp, b2.reshape(1, D), W3,
                            jnp.concatenate([as3, ad3], axis=0))
    outp, denp = _sc_gat(h, asd.reshape(NP), add_.reshape(NP), src, dst)

    out = _final(outp, denp, b3.reshape(1, D))
    return out[:n]


# speedup vs baseline: 32.7802x; 1.8156x over previous
import jax
import jax.numpy as jnp
from jax import lax
from jax.experimental import pallas as pl
from jax.experimental.pallas import tpu as pltpu
from jax.experimental.pallas import tpu_sc as plsc

f32 = jnp.float32
i32 = jnp.int32

D = 128
NP = 10240
NC, NS, L = 2, 16, 16
NW = NC * NS
CH = 64
EC = 162
EPW = EC * CH
EP = NW * EPW
BR = 1024
GRID = NP // BR
RPT = NP // NS


def _mm_att_body(x_ref, w_ref, av_ref, h_ref, as_ref, ad_ref):
    h = jnp.dot(x_ref[...], w_ref[...], preferred_element_type=f32)
    h_ref[...] = h
    as_ref[...] = jnp.sum(h * av_ref[0:1], axis=1).reshape(1, 1, BR)
    ad_ref[...] = jnp.sum(h * av_ref[1:2], axis=1).reshape(1, 1, BR)


_mm_att = pl.pallas_call(
    _mm_att_body,
    grid=(GRID,),
    in_specs=[
        pl.BlockSpec((BR, D), lambda i: (i, 0)),
        pl.BlockSpec((D, D), lambda i: (0, 0)),
        pl.BlockSpec((2, D), lambda i: (0, 0)),
    ],
    out_specs=[
        pl.BlockSpec((BR, D), lambda i: (i, 0)),
        pl.BlockSpec((1, 1, BR), lambda i: (i, 0, 0)),
        pl.BlockSpec((1, 1, BR), lambda i: (i, 0, 0)),
    ],
    out_shape=[
        jax.ShapeDtypeStruct((NP, D), f32),
        jax.ShapeDtypeStruct((GRID, 1, BR), f32),
        jax.ShapeDtypeStruct((GRID, 1, BR), f32),
    ],
)


def _comb_mm_body(op_ref, dp_ref, b_ref, w_ref, av_ref, h_ref, as_ref, ad_ref):
    z = op_ref[0] + op_ref[1]
    dtot = jnp.sum(dp_ref[...], axis=0)
    v = z / (dtot[:, None] + 1e-16) + b_ref[...]
    v = jnp.where(v > 0.0, v, jnp.exp(v) - 1.0)
    h = jnp.dot(v, w_ref[...], preferred_element_type=f32)
    h_ref[...] = h
    as_ref[...] = jnp.sum(h * av_ref[0:1], axis=1).reshape(1, 1, BR)
    ad_ref[...] = jnp.sum(h * av_ref[1:2], axis=1).reshape(1, 1, BR)


_comb_mm = pl.pallas_call(
    _comb_mm_body,
    grid=(GRID,),
    in_specs=[
        pl.BlockSpec((NC, BR, D), lambda i: (0, i, 0)),
        pl.BlockSpec((NW, BR), lambda i: (0, i)),
        pl.BlockSpec((1, D), lambda i: (0, 0)),
        pl.BlockSpec((D, D), lambda i: (0, 0)),
        pl.BlockSpec((2, D), lambda i: (0, 0)),
    ],
    out_specs=[
        pl.BlockSpec((BR, D), lambda i: (i, 0)),
        pl.BlockSpec((1, 1, BR), lambda i: (i, 0, 0)),
        pl.BlockSpec((1, 1, BR), lambda i: (i, 0, 0)),
    ],
    out_shape=[
        jax.ShapeDtypeStruct((NP, D), f32),
        jax.ShapeDtypeStruct((GRID, 1, BR), f32),
        jax.ShapeDtypeStruct((GRID, 1, BR), f32),
    ],
)


def _final_body(op_ref, dp_ref, b_ref, o_ref):
    z = op_ref[0] + op_ref[1]
    dtot = jnp.sum(dp_ref[...], axis=0)
    o_ref[...] = z / (dtot[:, None] + 1e-16) + b_ref[...]


_final = pl.pallas_call(
    _final_body,
    grid=(GRID,),
    in_specs=[
        pl.BlockSpec((NC, BR, D), lambda i: (0, i, 0)),
        pl.BlockSpec((NW, BR), lambda i: (0, i)),
        pl.BlockSpec((1, D), lambda i: (0, 0)),
    ],
    out_specs=pl.BlockSpec((BR, D), lambda i: (i, 0)),
    out_shape=jax.ShapeDtypeStruct((NP, D), f32),
)


def _sc_gat_body(h_hbm, as_hbm, ad_hbm, sd_hbm,
                 outp_hbm, denp_hbm,
                 as_v, ad_v, den_v, sd4, ex_c, rows2, acc_sh,
                 asem, isem, gsem):
    c = lax.axis_index("c")
    s = lax.axis_index("s")
    wid = c * NS + s

    acp1 = pltpu.async_copy(as_hbm, as_v, asem)
    acp2 = pltpu.async_copy(ad_hbm, ad_v, asem)
    icp0 = pltpu.async_copy(sd_hbm.at[wid, 0], sd4.at[0], isem)

    zeros16 = jnp.zeros((L,), f32)

    def _zero_den(i, carry):
        den_v[pl.ds(i * L, L)] = zeros16
        return carry

    lax.fori_loop(0, NP // L, _zero_den, 0)

    def _zero_rows(i, carry):
        for k in range(D // L):
            rows2[1, i, pl.ds(k * L, L)] = zeros16
        return carry

    lax.fori_loop(0, CH, _zero_rows, 0)

    icp0.wait()
    pltpu.async_copy(sd_hbm.at[wid, 1], sd4.at[1], isem)
    pltpu.async_copy(h_hbm.at[sd4.at[0, 0]], rows2.at[0], gsem)

    init_cps = [
        pltpu.async_copy(rows2.at[1], acc_sh.at[pl.ds(s * RPT + k * CH, CH)],
                         asem)
        for k in range(RPT // CH)
    ]
    acp1.wait()
    acp2.wait()
    for cp in init_cps:
        cp.wait()
    plsc.subcore_barrier()

    def _chunk(ci, carry):
        slot = lax.rem(ci, 4)
        nslot = lax.rem(ci + 1, 4)
        wslot = lax.rem(ci + 2, 4)
        b = lax.rem(ci, 2)
        nb = 1 - b

        pltpu.make_async_copy(sd_hbm.at[wid, 0], sd4.at[0], isem).wait()
        nci = jnp.minimum(ci + 2, EC - 1)
        pltpu.async_copy(sd_hbm.at[wid, nci], sd4.at[wslot], isem)

        pltpu.make_async_copy(h_hbm.at[sd4.at[0, 0]], rows2.at[0], gsem).wait()
        pltpu.async_copy(h_hbm.at[sd4.at[nslot, 0]], rows2.at[nb], gsem)

        def _ex16(j, carry2):
            sl = pl.ds(j * L, L)
            sv = sd4[slot, 0, sl]
            dv = sd4[slot, 1, sl]
            a = plsc.load_gather(as_v, [sv]) + plsc.load_gather(ad_v, [dv])
            a = jnp.where(a >= 0.0, a, 0.2 * a)
            e = jnp.exp(a)
            ex_c[sl] = e
            plsc.addupdate_scatter(den_v, [dv], e)
            return carry2

        lax.fori_loop(0, CH // L, _ex16, 0)

        def _scale(j, carry2):
            es = plsc.load_gather(ex_c, [jnp.full((L,), j, i32)])
            for k in range(D // L):
                sl = pl.ds(k * L, L)
                rows2[b, j, sl] = rows2[b, j, sl] * es
            return carry2

        lax.fori_loop(0, CH, _scale, 0)
        pltpu.sync_copy(rows2.at[b], acc_sh.at[sd4.at[slot, 1]], add=True)
        return carry

    lax.fori_loop(0, EC, _chunk, 0)

    pltpu.make_async_copy(sd_hbm.at[wid, 0], sd4.at[0], isem).wait()
    pltpu.make_async_copy(h_hbm.at[sd4.at[0, 0]], rows2.at[0], gsem).wait()

    plsc.subcore_barrier()
    pltpu.sync_copy(den_v, denp_hbm.at[wid])
    base = s * RPT
    pltpu.sync_copy(acc_sh.at[pl.ds(base, RPT)],
                    outp_hbm.at[c, pl.ds(base, RPT)])


_sc_gat = pl.kernel(
    _sc_gat_body,
    out_type=[
        jax.ShapeDtypeStruct((NC, NP, D), f32),
        jax.ShapeDtypeStruct((NW, NP), f32),
    ],
    mesh=plsc.VectorSubcoreMesh(core_axis_name="c", subcore_axis_name="s",
                                num_cores=NC, num_subcores=NS),
    compiler_params=pltpu.CompilerParams(needs_layout_passes=False),
    scratch_types=[
        pltpu.VMEM((NP,), f32),
        pltpu.VMEM((NP,), f32),
        pltpu.VMEM((NP,), f32),
        pltpu.VMEM((4, 2, CH), i32),
        pltpu.VMEM((CH,), f32),
        pltpu.VMEM((2, CH, D), f32),
        pltpu.VMEM_SHARED((NP, D), f32),
        pltpu.SemaphoreType.DMA,
        pltpu.SemaphoreType.DMA,
        pltpu.SemaphoreType.DMA,
    ],
)


def kernel(x, edge_index, batch, W1, as1, ad1, b1, W2, as2, ad2, b2,
           W3, as3, ad3, b3):
    n, d = x.shape
    e = edge_index.shape[1]
    xp = jnp.pad(x, ((0, NP - n), (0, 0)))
    loop = jnp.arange(n, dtype=i32)
    pad_e = jnp.full((EP - e - n,), n + 1, i32)
    src = jnp.concatenate([edge_index[0], loop, pad_e]).reshape(NW, EC, CH)
    dst = jnp.concatenate([edge_index[1], loop, pad_e]).reshape(NW, EC, CH)
    sd = jnp.stack([src, dst], axis=2)

    h, asd, add_ = _mm_att(xp, W1, jnp.concatenate([as1, ad1], axis=0))
    outp, denp = _sc_gat(h, asd.reshape(NP), add_.reshape(NP), sd)

    h, asd, add_ = _comb_mm(outp, denp, b1.reshape(1, D), W2,
                            jnp.concatenate([as2, ad2], axis=0))
    outp, denp = _sc_gat(h, asd.reshape(NP), add_.reshape(NP), sd)

    h, asd, add_ = _comb_mm(outp, denp, b2.reshape(1, D), W3,
                            jnp.concatenate([as3, ad3], axis=0))
    outp, denp = _sc_gat(h, asd.reshape(NP), add_.reshape(NP), sd)

    out = _final(outp, denp, b3.reshape(1, D))
    return out[:n]


# speedup vs baseline: 35.2742x; 1.0761x over previous
import jax
import jax.numpy as jnp
from jax import lax
from jax.experimental import pallas as pl
from jax.experimental.pallas import tpu as pltpu
from jax.experimental.pallas import tpu_sc as plsc

f32 = jnp.float32
i32 = jnp.int32

D = 128
NP = 10240
NC, NS, L = 2, 16, 16
NW = NC * NS
CH = 64
EC = 162
EPW = EC * CH
EP = NW * EPW
BR = 1024
GRID = NP // BR
RPT = NP // NS


def _mm_att_body(x_ref, w_ref, av_ref, h_ref, as_ref, ad_ref):
    h = jnp.dot(x_ref[...], w_ref[...], preferred_element_type=f32)
    h_ref[...] = h
    as_ref[...] = jnp.sum(h * av_ref[0:1], axis=1).reshape(1, 1, BR)
    ad_ref[...] = jnp.sum(h * av_ref[1:2], axis=1).reshape(1, 1, BR)


_mm_att = pl.pallas_call(
    _mm_att_body,
    grid=(GRID,),
    in_specs=[
        pl.BlockSpec((BR, D), lambda i: (i, 0)),
        pl.BlockSpec((D, D), lambda i: (0, 0)),
        pl.BlockSpec((2, D), lambda i: (0, 0)),
    ],
    out_specs=[
        pl.BlockSpec((BR, D), lambda i: (i, 0)),
        pl.BlockSpec((1, 1, BR), lambda i: (i, 0, 0)),
        pl.BlockSpec((1, 1, BR), lambda i: (i, 0, 0)),
    ],
    out_shape=[
        jax.ShapeDtypeStruct((NP, D), f32),
        jax.ShapeDtypeStruct((GRID, 1, BR), f32),
        jax.ShapeDtypeStruct((GRID, 1, BR), f32),
    ],
)


def _comb_mm_body(op_ref, dp_ref, b_ref, w_ref, av_ref, h_ref, as_ref, ad_ref):
    z = op_ref[0] + op_ref[1]
    dtot = jnp.sum(dp_ref[...], axis=0)
    v = z / (dtot[:, None] + 1e-16) + b_ref[...]
    v = jnp.where(v > 0.0, v, jnp.exp(v) - 1.0)
    h = jnp.dot(v, w_ref[...], preferred_element_type=f32)
    h_ref[...] = h
    as_ref[...] = jnp.sum(h * av_ref[0:1], axis=1).reshape(1, 1, BR)
    ad_ref[...] = jnp.sum(h * av_ref[1:2], axis=1).reshape(1, 1, BR)


_comb_mm = pl.pallas_call(
    _comb_mm_body,
    grid=(GRID,),
    in_specs=[
        pl.BlockSpec((NC, BR, D), lambda i: (0, i, 0)),
        pl.BlockSpec((NW, BR), lambda i: (0, i)),
        pl.BlockSpec((1, D), lambda i: (0, 0)),
        pl.BlockSpec((D, D), lambda i: (0, 0)),
        pl.BlockSpec((2, D), lambda i: (0, 0)),
    ],
    out_specs=[
        pl.BlockSpec((BR, D), lambda i: (i, 0)),
        pl.BlockSpec((1, 1, BR), lambda i: (i, 0, 0)),
        pl.BlockSpec((1, 1, BR), lambda i: (i, 0, 0)),
    ],
    out_shape=[
        jax.ShapeDtypeStruct((NP, D), f32),
        jax.ShapeDtypeStruct((GRID, 1, BR), f32),
        jax.ShapeDtypeStruct((GRID, 1, BR), f32),
    ],
)


def _final_body(op_ref, dp_ref, b_ref, o_ref):
    z = op_ref[0] + op_ref[1]
    dtot = jnp.sum(dp_ref[...], axis=0)
    o_ref[...] = z / (dtot[:, None] + 1e-16) + b_ref[...]


_final = pl.pallas_call(
    _final_body,
    grid=(GRID,),
    in_specs=[
        pl.BlockSpec((NC, BR, D), lambda i: (0, i, 0)),
        pl.BlockSpec((NW, BR), lambda i: (0, i)),
        pl.BlockSpec((1, D), lambda i: (0, 0)),
    ],
    out_specs=pl.BlockSpec((BR, D), lambda i: (i, 0)),
    out_shape=jax.ShapeDtypeStruct((NP, D), f32),
)


def _splat(vec, j):
    idx = jnp.full((L, 1), j, i32)
    dnums = lax.GatherDimensionNumbers(offset_dims=(),
                                       collapsed_slice_dims=(0,),
                                       start_index_map=(0,))
    return lax.gather(vec, idx, dnums, (1,),
                      mode=lax.GatherScatterMode.PROMISE_IN_BOUNDS)


def _sc_gat_body(h_hbm, as_hbm, ad_hbm, sd_hbm,
                 outp_hbm, denp_hbm,
                 as_v, ad_v, den_v, sd4, rows2, acc_sh,
                 asem, isem, gsem, ssem):
    c = lax.axis_index("c")
    s = lax.axis_index("s")
    wid = c * NS + s

    acp1 = pltpu.async_copy(as_hbm, as_v, asem)
    acp2 = pltpu.async_copy(ad_hbm, ad_v, asem)
    icp0 = pltpu.async_copy(sd_hbm.at[wid, 0], sd4.at[0], isem)

    zeros16 = jnp.zeros((L,), f32)

    def _zero_den(i, carry):
        den_v[pl.ds(i * L, L)] = zeros16
        return carry

    lax.fori_loop(0, NP // L, _zero_den, 0)

    def _zero_rows(i, carry):
        for k in range(D // L):
            rows2[1, i, pl.ds(k * L, L)] = zeros16
        return carry

    lax.fori_loop(0, CH, _zero_rows, 0)

    icp0.wait()
    pltpu.async_copy(sd_hbm.at[wid, 1], sd4.at[1], isem)
    pltpu.async_copy(h_hbm.at[sd4.at[0, 0]], rows2.at[0], gsem)

    init_cps = [
        pltpu.async_copy(rows2.at[1], acc_sh.at[pl.ds(s * RPT + k * CH, CH)],
                         asem)
        for k in range(RPT // CH)
    ]
    acp1.wait()
    acp2.wait()
    for cp in init_cps:
        cp.wait()
    plsc.subcore_barrier()

    def _chunk(ci, carry):
        slot = lax.rem(ci, 4)
        nslot = lax.rem(ci + 1, 4)
        wslot = lax.rem(ci + 2, 4)
        b = lax.rem(ci, 2)
        nb = 1 - b

        pltpu.make_async_copy(sd_hbm.at[wid, 0], sd4.at[0], isem).wait()
        nci = jnp.minimum(ci + 2, EC - 1)
        pltpu.async_copy(sd_hbm.at[wid, nci], sd4.at[wslot], isem)

        pltpu.make_async_copy(h_hbm.at[sd4.at[0, 0]], rows2.at[0], gsem).wait()

        @pl.when(ci != 0)
        def _wait_prev_scatter():
            pltpu.make_async_copy(rows2.at[0], acc_sh.at[pl.ds(0, CH)],
                                  ssem).wait()

        pltpu.async_copy(h_hbm.at[sd4.at[nslot, 0]], rows2.at[nb], gsem)

        def _body16(g, carry2):
            sl = pl.ds(g * L, L)
            sv = sd4[slot, 0, sl]
            dv = sd4[slot, 1, sl]
            a = plsc.load_gather(as_v, [sv]) + plsc.load_gather(ad_v, [dv])
            a = jnp.where(a >= 0.0, a, 0.2 * a)
            ex16 = jnp.exp(a)
            plsc.addupdate_scatter(den_v, [dv], ex16)

            def _scale(jj, carry3):
                es = _splat(ex16, jj)
                j = g * L + jj
                for k in range(D // L):
                    ksl = pl.ds(k * L, L)
                    rows2[b, j, ksl] = rows2[b, j, ksl] * es
                return carry3

            lax.fori_loop(0, L, _scale, 0)
            return carry2

        lax.fori_loop(0, CH // L, _body16, 0)
        pltpu.async_copy(rows2.at[b], acc_sh.at[sd4.at[slot, 1]], ssem,
                         add=True)
        return carry

    lax.fori_loop(0, EC, _chunk, 0)

    pltpu.make_async_copy(sd_hbm.at[wid, 0], sd4.at[0], isem).wait()
    pltpu.make_async_copy(h_hbm.at[sd4.at[0, 0]], rows2.at[0], gsem).wait()
    pltpu.make_async_copy(rows2.at[0], acc_sh.at[pl.ds(0, CH)], ssem).wait()

    plsc.subcore_barrier()
    pltpu.sync_copy(den_v, denp_hbm.at[wid])
    base = s * RPT
    pltpu.sync_copy(acc_sh.at[pl.ds(base, RPT)],
                    outp_hbm.at[c, pl.ds(base, RPT)])


_sc_gat = pl.kernel(
    _sc_gat_body,
    out_type=[
        jax.ShapeDtypeStruct((NC, NP, D), f32),
        jax.ShapeDtypeStruct((NW, NP), f32),
    ],
    mesh=plsc.VectorSubcoreMesh(core_axis_name="c", subcore_axis_name="s",
                                num_cores=NC, num_subcores=NS),
    compiler_params=pltpu.CompilerParams(needs_layout_passes=False),
    scratch_types=[
        pltpu.VMEM((NP,), f32),
        pltpu.VMEM((NP,), f32),
        pltpu.VMEM((NP,), f32),
        pltpu.VMEM((4, 2, CH), i32),
        pltpu.VMEM((2, CH, D), f32),
        pltpu.VMEM_SHARED((NP, D), f32),
        pltpu.SemaphoreType.DMA,
        pltpu.SemaphoreType.DMA,
        pltpu.SemaphoreType.DMA,
        pltpu.SemaphoreType.DMA,
    ],
)


def kernel(x, edge_index, batch, W1, as1, ad1, b1, W2, as2, ad2, b2,
           W3, as3, ad3, b3):
    n, d = x.shape
    e = edge_index.shape[1]
    xp = jnp.pad(x, ((0, NP - n), (0, 0)))
    loop = jnp.arange(n, dtype=i32)
    pad_e = jnp.full((EP - e - n,), n + 1, i32)
    src = jnp.concatenate([edge_index[0], loop, pad_e]).reshape(NW, EC, CH)
    dst = jnp.concatenate([edge_index[1], loop, pad_e]).reshape(NW, EC, CH)
    sd = jnp.stack([src, dst], axis=2)

    h, asd, add_ = _mm_att(xp, W1, jnp.concatenate([as1, ad1], axis=0))
    outp, denp = _sc_gat(h, asd.reshape(NP), add_.reshape(NP), sd)

    h, asd, add_ = _comb_mm(outp, denp, b1.reshape(1, D), W2,
                            jnp.concatenate([as2, ad2], axis=0))
    outp, denp = _sc_gat(h, asd.reshape(NP), add_.reshape(NP), sd)

    h, asd, add_ = _comb_mm(outp, denp, b2.reshape(1, D), W3,
                            jnp.concatenate([as3, ad3], axis=0))
    outp, denp = _sc_gat(h, asd.reshape(NP), add_.reshape(NP), sd)

    out = _final(outp, denp, b3.reshape(1, D))
    return out[:n]


# speedup vs baseline: 37.7938x; 1.0714x over previous
import jax
import jax.numpy as jnp
from jax import lax
from jax.experimental import pallas as pl
from jax.experimental.pallas import tpu as pltpu
from jax.experimental.pallas import tpu_sc as plsc

f32 = jnp.float32
i32 = jnp.int32

D = 128
NP = 10240
NC, NS, L = 2, 16, 16
NW = NC * NS
CH = 64
EC = 162
EPW = EC * CH
EP = NW * EPW
BR = 1024
GRID = NP // BR
RPT = NP // NS
NPD = 10016


def _mm_att_body(x_ref, w_ref, av_ref, h_ref, as_ref, ad_ref):
    h = jnp.dot(x_ref[...], w_ref[...], preferred_element_type=f32)
    h_ref[...] = h
    as_ref[...] = jnp.sum(h * av_ref[0:1], axis=1).reshape(1, 1, BR)
    ad_ref[...] = jnp.sum(h * av_ref[1:2], axis=1).reshape(1, 1, BR)


_mm_att = pl.pallas_call(
    _mm_att_body,
    grid=(GRID,),
    in_specs=[
        pl.BlockSpec((BR, D), lambda i: (i, 0)),
        pl.BlockSpec((D, D), lambda i: (0, 0)),
        pl.BlockSpec((2, D), lambda i: (0, 0)),
    ],
    out_specs=[
        pl.BlockSpec((BR, D), lambda i: (i, 0)),
        pl.BlockSpec((1, 1, BR), lambda i: (i, 0, 0)),
        pl.BlockSpec((1, 1, BR), lambda i: (i, 0, 0)),
    ],
    out_shape=[
        jax.ShapeDtypeStruct((NP, D), f32),
        jax.ShapeDtypeStruct((GRID, 1, BR), f32),
        jax.ShapeDtypeStruct((GRID, 1, BR), f32),
    ],
)


def _comb_mm_body(op_ref, dp_ref, b_ref, w_ref, av_ref, h_ref, as_ref, ad_ref):
    z = op_ref[0] + op_ref[1]
    dtot = jnp.sum(dp_ref[...], axis=0)
    v = z / (dtot[:, None] + 1e-16) + b_ref[...]
    v = jnp.where(v > 0.0, v, jnp.exp(v) - 1.0)
    h = jnp.dot(v, w_ref[...], preferred_element_type=f32)
    h_ref[...] = h
    as_ref[...] = jnp.sum(h * av_ref[0:1], axis=1).reshape(1, 1, BR)
    ad_ref[...] = jnp.sum(h * av_ref[1:2], axis=1).reshape(1, 1, BR)


_comb_mm = pl.pallas_call(
    _comb_mm_body,
    grid=(GRID,),
    in_specs=[
        pl.BlockSpec((NC, BR, D), lambda i: (0, i, 0)),
        pl.BlockSpec((NC, BR), lambda i: (0, i)),
        pl.BlockSpec((1, D), lambda i: (0, 0)),
        pl.BlockSpec((D, D), lambda i: (0, 0)),
        pl.BlockSpec((2, D), lambda i: (0, 0)),
    ],
    out_specs=[
        pl.BlockSpec((BR, D), lambda i: (i, 0)),
        pl.BlockSpec((1, 1, BR), lambda i: (i, 0, 0)),
        pl.BlockSpec((1, 1, BR), lambda i: (i, 0, 0)),
    ],
    out_shape=[
        jax.ShapeDtypeStruct((NP, D), f32),
        jax.ShapeDtypeStruct((GRID, 1, BR), f32),
        jax.ShapeDtypeStruct((GRID, 1, BR), f32),
    ],
)


def _final_body(op_ref, dp_ref, b_ref, o_ref):
    z = op_ref[0] + op_ref[1]
    dtot = jnp.sum(dp_ref[...], axis=0)
    o_ref[...] = z / (dtot[:, None] + 1e-16) + b_ref[...]


_final = pl.pallas_call(
    _final_body,
    grid=(GRID,),
    in_specs=[
        pl.BlockSpec((NC, BR, D), lambda i: (0, i, 0)),
        pl.BlockSpec((NC, BR), lambda i: (0, i)),
        pl.BlockSpec((1, D), lambda i: (0, 0)),
    ],
    out_specs=pl.BlockSpec((BR, D), lambda i: (i, 0)),
    out_shape=jax.ShapeDtypeStruct((NP, D), f32),
)


def _splat(vec, j):
    idx = jnp.full((L, 1), j, i32)
    dnums = lax.GatherDimensionNumbers(offset_dims=(),
                                       collapsed_slice_dims=(0,),
                                       start_index_map=(0,))
    return lax.gather(vec, idx, dnums, (1,),
                      mode=lax.GatherScatterMode.PROMISE_IN_BOUNDS)


def _sc_gat_body(h_hbm, as_hbm, ad_hbm, sd_hbm,
                 outp_hbm, denp_hbm,
                 as_v, ad_v, exb, zb, sd4, rows3, den_sh, acc_sh,
                 asem, isem, g0sem, g1sem, ssem):
    c = lax.axis_index("c")
    s = lax.axis_index("s")
    wid = c * NS + s
    gsems = (g0sem, g1sem)

    acp1 = pltpu.async_copy(as_hbm.at[pl.ds(0, NPD)], as_v, asem)
    acp2 = pltpu.async_copy(ad_hbm.at[pl.ds(0, NPD)], ad_v, asem)
    icp0 = pltpu.async_copy(sd_hbm.at[wid, 0], sd4.at[0], isem)

    zeros16 = jnp.zeros((L,), f32)

    def _zero_zb(i, carry):
        zb[pl.ds(i * L, L)] = zeros16
        return carry

    lax.fori_loop(0, RPT // L, _zero_zb, 0)
    icp0.wait()
    pltpu.async_copy(sd_hbm.at[wid, 1], sd4.at[1], isem)

    def _zero_rows(i, carry):
        for k in range(D // L):
            rows3[2, i, pl.ds(k * L, L)] = zeros16
        return carry

    lax.fori_loop(0, CH, _zero_rows, 0)
    pltpu.make_async_copy(sd_hbm.at[wid, 0], sd4.at[0], isem).wait()
    pltpu.async_copy(sd_hbm.at[wid, 2], sd4.at[2], isem)
    pltpu.async_copy(h_hbm.at[sd4.at[0, 0]], rows3.at[0], g0sem)
    pltpu.async_copy(h_hbm.at[sd4.at[1, 0]], rows3.at[1], g1sem)

    init_cps = [
        pltpu.async_copy(rows3.at[2], acc_sh.at[pl.ds(s * RPT + k * CH, CH)],
                         asem)
        for k in range(RPT // CH)
    ]
    init_cps.append(
        pltpu.async_copy(zb, den_sh.at[pl.ds(s * RPT, RPT)], asem))
    acp1.wait()
    acp2.wait()
    for cp in init_cps:
        cp.wait()
    plsc.subcore_barrier()

    def _one_chunk(ci, par):
        gsem = gsems[par]
        slot4 = lax.rem(ci, 4)
        w4 = lax.rem(ci + 3, 4)
        g4 = lax.rem(ci + 2, 4)
        r3 = lax.rem(ci, 3)
        gr3 = lax.rem(ci + 2, 3)
        eb = lax.rem(ci, 2)

        pltpu.make_async_copy(sd_hbm.at[wid, 0], sd4.at[0], isem).wait()

        @pl.when(ci != 0)
        def _wait_prev_scatter():
            pltpu.make_async_copy(rows3.at[0], acc_sh.at[pl.ds(0, CH)],
                                  ssem).wait()
            pltpu.make_async_copy(exb.at[0], den_sh.at[pl.ds(0, CH)],
                                  ssem).wait()

        pltpu.async_copy(sd_hbm.at[wid, jnp.minimum(ci + 3, EC - 1)],
                         sd4.at[w4], isem)
        pltpu.make_async_copy(h_hbm.at[sd4.at[0, 0]], rows3.at[0], gsem).wait()
        pltpu.async_copy(h_hbm.at[sd4.at[g4, 0]], rows3.at[gr3], gsem)

        def _body16(g, carry2):
            sl = pl.ds(g * L, L)
            sv = sd4[slot4, 0, sl]
            dv = sd4[slot4, 1, sl]
            a = plsc.load_gather(as_v, [sv]) + plsc.load_gather(ad_v, [dv])
            a = jnp.where(a >= 0.0, a, 0.2 * a)
            ex16 = jnp.exp(a)
            exb[eb, sl] = ex16

            def _scale(jj, carry3):
                es = _splat(ex16, jj)
                j = g * L + jj
                for k in range(D // L):
                    ksl = pl.ds(k * L, L)
                    rows3[r3, j, ksl] = rows3[r3, j, ksl] * es
                return carry3

            lax.fori_loop(0, L, _scale, 0)
            return carry2

        lax.fori_loop(0, CH // L, _body16, 0)
        pltpu.async_copy(rows3.at[r3], acc_sh.at[sd4.at[slot4, 1]], ssem,
                         add=True)
        pltpu.async_copy(exb.at[eb], den_sh.at[sd4.at[slot4, 1]], ssem,
                         add=True)

    def _chunk2(tt, carry):
        _one_chunk(2 * tt, 0)
        _one_chunk(2 * tt + 1, 1)
        return carry

    lax.fori_loop(0, EC // 2, _chunk2, 0)

    pltpu.make_async_copy(sd_hbm.at[wid, 0], sd4.at[0], isem).wait()
    pltpu.make_async_copy(h_hbm.at[sd4.at[0, 0]], rows3.at[0], g0sem).wait()
    pltpu.make_async_copy(h_hbm.at[sd4.at[0, 0]], rows3.at[0], g1sem).wait()
    pltpu.make_async_copy(rows3.at[0], acc_sh.at[pl.ds(0, CH)], ssem).wait()
    pltpu.make_async_copy(exb.at[0], den_sh.at[pl.ds(0, CH)], ssem).wait()

    plsc.subcore_barrier()
    pltpu.sync_copy(den_sh.at[pl.ds(s * RPT, RPT)], denp_hbm.at[c, s])
    base = s * RPT
    pltpu.sync_copy(acc_sh.at[pl.ds(base, RPT)],
                    outp_hbm.at[c, pl.ds(base, RPT)])


_sc_gat = pl.kernel(
    _sc_gat_body,
    out_type=[
        jax.ShapeDtypeStruct((NC, NP, D), f32),
        jax.ShapeDtypeStruct((NC, NS, RPT), f32),
    ],
    mesh=plsc.VectorSubcoreMesh(core_axis_name="c", subcore_axis_name="s",
                                num_cores=NC, num_subcores=NS),
    compiler_params=pltpu.CompilerParams(needs_layout_passes=False),
    scratch_types=[
        pltpu.VMEM((NPD,), f32),
        pltpu.VMEM((NPD,), f32),
        pltpu.VMEM((2, CH), f32),
        pltpu.VMEM((RPT,), f32),
        pltpu.VMEM((4, 2, CH), i32),
        pltpu.VMEM((3, CH, D), f32),
        pltpu.VMEM_SHARED((NP,), f32),
        pltpu.VMEM_SHARED((NP, D), f32),
        pltpu.SemaphoreType.DMA,
        pltpu.SemaphoreType.DMA,
        pltpu.SemaphoreType.DMA,
        pltpu.SemaphoreType.DMA,
        pltpu.SemaphoreType.DMA,
    ],
)


def kernel(x, edge_index, batch, W1, as1, ad1, b1, W2, as2, ad2, b2,
           W3, as3, ad3, b3):
    n, d = x.shape
    e = edge_index.shape[1]
    xp = jnp.pad(x, ((0, NP - n), (0, 0)))
    loop = jnp.arange(n, dtype=i32)
    pad_e = jnp.full((EP - e - n,), n + 1, i32)
    src = jnp.concatenate([edge_index[0], loop, pad_e]).reshape(NW, EC, CH)
    dst = jnp.concatenate([edge_index[1], loop, pad_e]).reshape(NW, EC, CH)
    sd = jnp.stack([src, dst], axis=2)

    h, asd, add_ = _mm_att(xp, W1, jnp.concatenate([as1, ad1], axis=0))
    outp, denp = _sc_gat(h, asd.reshape(NP), add_.reshape(NP), sd)

    h, asd, add_ = _comb_mm(outp, denp.reshape(NC, NP), b1.reshape(1, D), W2,
                            jnp.concatenate([as2, ad2], axis=0))
    outp, denp = _sc_gat(h, asd.reshape(NP), add_.reshape(NP), sd)

    h, asd, add_ = _comb_mm(outp, denp.reshape(NC, NP), b2.reshape(1, D), W3,
                            jnp.concatenate([as3, ad3], axis=0))
    outp, denp = _sc_gat(h, asd.reshape(NP), add_.reshape(NP), sd)

    out = _final(outp, denp.reshape(NC, NP), b3.reshape(1, D))
    return out[:n]
